# baseline (speedup 1.0000x reference)
import functools

import jax
import jax.numpy as jnp
from jax import lax
from jax.experimental import pallas as pl
from jax.experimental.pallas import tpu as pltpu
from jax.experimental.pallas import tpu_sc as plsc

B = 4096
H = 128
NT = 73
ROWS = B * NT
NW = 32
RPT = ROWS // NW
CHUNK = 128
NCH = RPT // CHUNK
NPAIR = (NCH - 1) // 2

OFF_PIECE = 0
OFF_SIDE = 13
OFF_CASTLE = 15
OFF_NOC = 19
OFF_POS = 20
OFF_NOEP = 84
OFF_HALF = 85
OFF_FULL = 185
OFF_REP = 441
ZERO_ROW = 444
TROWS = 448


def _prep_body(piece_ref, side_ref, castle_ref, ep_ref, hm_ref, fm_ref, rep_ref,
               idxa_ref, idxb_ref):
    board_a = piece_ref[...] + OFF_PIECE
    side_a = side_ref[...] + OFF_SIDE
    cast_a = jnp.where(castle_ref[...] > 0,
                       lax.broadcasted_iota(jnp.int32, (B, 4), 1) + OFF_CASTLE,
                       OFF_NOC)
    ep = ep_ref[...]
    ep_a = jnp.where(ep < 64, ep + OFF_POS, OFF_NOEP)
    hm_a = jnp.clip(hm_ref[...], 0, 99) + OFF_HALF
    fm_a = jnp.clip(fm_ref[...], 1, 256) - 1 + OFF_FULL
    rep_a = jnp.clip(rep_ref[...] - 1, 0, 2) + OFF_REP
    idxa_ref[...] = jnp.concatenate(
        [board_a, side_a, cast_a, ep_a, hm_a, fm_a, rep_a], axis=1)
    board_b = lax.broadcasted_iota(jnp.int32, (B, 64), 1) + OFF_POS
    idxb_ref[...] = jnp.concatenate(
        [board_b, jnp.full((B, 9), ZERO_ROW, jnp.int32)], axis=1)


_prep = pl.pallas_call(
    _prep_body,
    out_shape=(jax.ShapeDtypeStruct((B, NT), jnp.int32),
               jax.ShapeDtypeStruct((B, NT), jnp.int32)),
)


def _sc_body(table_hbm, idxa_hbm, idxb_hbm, out_hbm,
             table_v, idxa_v, idxb_v, rows0, rows1, ss0, ss1):
    wid = lax.axis_index("s") * 2 + lax.axis_index("c")
    base0 = wid * RPT

    pltpu.sync_copy(table_hbm, table_v)
    pltpu.sync_copy(idxa_hbm.at[pl.ds(base0, RPT)], idxa_v)
    pltpu.sync_copy(idxb_hbm.at[pl.ds(base0, RPT)], idxb_v)

    lane_h = jnp.arange(16, dtype=jnp.int32) * H

    def compute_chunk(c, buf):
        def block(k, carry):
            r0 = c * CHUNK + k * 16
            addr_a = idxa_v[pl.ds(r0, 16)] * H
            addr_b = idxb_v[pl.ds(r0, 16)] * H
            addr_o = k * (16 * H) + lane_h
            for _ in range(H):
                va = plsc.load_gather(table_v, [addr_a])
                vb = plsc.load_gather(table_v, [addr_b])
                plsc.store_scatter(buf, [addr_o], va + vb)
                addr_a = addr_a + 1
                addr_b = addr_b + 1
                addr_o = addr_o + 1
            return carry
        lax.fori_loop(0, CHUNK // 16, block, 0)

    def chunk(c, carry):
        compute_chunk(c, rows0)
        pltpu.async_copy(
            rows0, out_hbm.at[pl.ds((base0 + c * CHUNK) * H, CHUNK * H)],
            ss0).wait()
        return carry

    lax.fori_loop(0, NCH, chunk, 0)


_sc_gather = pl.kernel(
    _sc_body,
    out_type=jax.ShapeDtypeStruct((ROWS * H,), jnp.float32),
    mesh=plsc.VectorSubcoreMesh(core_axis_name="c", subcore_axis_name="s"),
    compiler_params=pltpu.CompilerParams(needs_layout_passes=False),
    scratch_types=(
        pltpu.VMEM((TROWS * H,), jnp.float32),
        pltpu.VMEM((RPT,), jnp.int32),
        pltpu.VMEM((RPT,), jnp.int32),
        pltpu.VMEM((CHUNK * H,), jnp.float32),
        pltpu.VMEM((CHUNK * H,), jnp.float32),
        pltpu.SemaphoreType.DMA,
        pltpu.SemaphoreType.DMA,
    ),
)


def kernel(piece_indices, side_idx, castling_flags, en_passant_idx, halfmove,
           fullmove, repetitions, side_embed, castling_embed_K, castling_embed_Q,
           castling_embed_k, castling_embed_q, no_castling_embed, piece_embed,
           no_en_passant_embed, half_move_embed, full_move_embed,
           repetition_embed, pos_embed):
    i32 = jnp.int32
    idxa, idxb = _prep(
        piece_indices.astype(i32),
        side_idx.astype(i32).reshape(B, 1),
        castling_flags.astype(i32),
        en_passant_idx.astype(i32).reshape(B, 1),
        halfmove.astype(i32).reshape(B, 1),
        fullmove.astype(i32).reshape(B, 1),
        repetitions.astype(i32).reshape(B, 1))

    table = jnp.concatenate([
        piece_embed,
        side_embed,
        castling_embed_K.reshape(1, H),
        castling_embed_Q.reshape(1, H),
        castling_embed_k.reshape(1, H),
        castling_embed_q.reshape(1, H),
        no_castling_embed.reshape(1, H),
        pos_embed,
        no_en_passant_embed.reshape(1, H),
        half_move_embed,
        full_move_embed,
        repetition_embed,
        jnp.zeros((TROWS - ZERO_ROW, H), jnp.float32),
    ], axis=0)

    flat = _sc_gather(table.reshape(TROWS * H),
                      idxa.reshape(ROWS), idxb.reshape(ROWS))
    return flat.reshape(B, NT, H)


# speedup vs baseline: 3.2983x; 3.2983x over previous
import functools

import jax
import jax.numpy as jnp
from jax import lax
from jax.experimental import pallas as pl
from jax.experimental.pallas import tpu as pltpu
from jax.experimental.pallas import tpu_sc as plsc

B = 4096
H = 128
NT = 73
ROWS = B * NT
NW = 32
RPT = ROWS // NW
CHUNK = 128
NCH = RPT // CHUNK
NPAIR = (NCH - 1) // 2

OFF_PIECE = 0
OFF_SIDE = 13
OFF_CASTLE = 15
OFF_NOC = 19
OFF_POS = 20
OFF_NOEP = 84
OFF_HALF = 85
OFF_FULL = 185
OFF_REP = 441
ZERO_ROW = 444
TROWS = 448


def _prep_body(piece_ref, side_ref, castle_ref, ep_ref, hm_ref, fm_ref, rep_ref,
               idxa_ref, idxb_ref):
    board_a = piece_ref[...] + OFF_PIECE
    side_a = side_ref[...] + OFF_SIDE
    cast_a = jnp.where(castle_ref[...] > 0,
                       lax.broadcasted_iota(jnp.int32, (B, 4), 1) + OFF_CASTLE,
                       OFF_NOC)
    ep = ep_ref[...]
    ep_a = jnp.where(ep < 64, ep + OFF_POS, OFF_NOEP)
    hm_a = jnp.clip(hm_ref[...], 0, 99) + OFF_HALF
    fm_a = jnp.clip(fm_ref[...], 1, 256) - 1 + OFF_FULL
    rep_a = jnp.clip(rep_ref[...] - 1, 0, 2) + OFF_REP
    idxa_ref[...] = jnp.concatenate(
        [board_a, side_a, cast_a, ep_a, hm_a, fm_a, rep_a], axis=1)
    board_b = lax.broadcasted_iota(jnp.int32, (B, 64), 1) + OFF_POS
    idxb_ref[...] = jnp.concatenate(
        [board_b, jnp.full((B, 9), ZERO_ROW, jnp.int32)], axis=1)


_prep = pl.pallas_call(
    _prep_body,
    out_shape=(jax.ShapeDtypeStruct((B, NT), jnp.int32),
               jax.ShapeDtypeStruct((B, NT), jnp.int32)),
)


def _sc_body(table_hbm, idxa_hbm, idxb_hbm, out_hbm,
             table_v, idxa_v, idxb_v, rows0, rows1, ss0, ss1):
    wid = lax.axis_index("s") * 2 + lax.axis_index("c")
    base0 = wid * RPT

    pltpu.sync_copy(table_hbm, table_v)
    pltpu.sync_copy(idxa_hbm.at[pl.ds(base0, RPT)], idxa_v)
    pltpu.sync_copy(idxb_hbm.at[pl.ds(base0, RPT)], idxb_v)

    def compute_chunk(c, buf):
        def block(k, carry):
            r0 = c * CHUNK + k * 16
            va = idxa_v[pl.ds(r0, 16)] * H
            vb = idxb_v[pl.ds(r0, 16)] * H
            for l in range(16):
                a = va[l]
                b = vb[l]
                o = (k * 16 + l) * H
                for j in range(H // 16):
                    buf[pl.ds(o + j * 16, 16)] = (
                        table_v[pl.ds(a + j * 16, 16)]
                        + table_v[pl.ds(b + j * 16, 16)])
            return carry
        lax.fori_loop(0, CHUNK // 16, block, 0)

    def chunk(c, carry):
        base = base0 + c * CHUNK
        compute_chunk(c, rows0)
        pltpu.async_copy(
            rows0, out_hbm.at[pl.ds(base * H, CHUNK * H)], ss0).wait()
        return carry

    lax.fori_loop(0, NCH, chunk, 0)


_sc_gather = pl.kernel(
    _sc_body,
    out_type=jax.ShapeDtypeStruct((ROWS * H,), jnp.float32),
    mesh=plsc.VectorSubcoreMesh(core_axis_name="c", subcore_axis_name="s"),
    compiler_params=pltpu.CompilerParams(needs_layout_passes=False),
    scratch_types=(
        pltpu.VMEM((TROWS * H,), jnp.float32),
        pltpu.VMEM((RPT,), jnp.int32),
        pltpu.VMEM((RPT,), jnp.int32),
        pltpu.VMEM((CHUNK * H,), jnp.float32),
        pltpu.VMEM((CHUNK * H,), jnp.float32),
        pltpu.SemaphoreType.DMA,
        pltpu.SemaphoreType.DMA,
    ),
)


def kernel(piece_indices, side_idx, castling_flags, en_passant_idx, halfmove,
           fullmove, repetitions, side_embed, castling_embed_K, castling_embed_Q,
           castling_embed_k, castling_embed_q, no_castling_embed, piece_embed,
           no_en_passant_embed, half_move_embed, full_move_embed,
           repetition_embed, pos_embed):
    i32 = jnp.int32
    idxa, idxb = _prep(
        piece_indices.astype(i32),
        side_idx.astype(i32).reshape(B, 1),
        castling_flags.astype(i32),
        en_passant_idx.astype(i32).reshape(B, 1),
        halfmove.astype(i32).reshape(B, 1),
        fullmove.astype(i32).reshape(B, 1),
        repetitions.astype(i32).reshape(B, 1))

    table = jnp.concatenate([
        piece_embed,
        side_embed,
        castling_embed_K.reshape(1, H),
        castling_embed_Q.reshape(1, H),
        castling_embed_k.reshape(1, H),
        castling_embed_q.reshape(1, H),
        no_castling_embed.reshape(1, H),
        pos_embed,
        no_en_passant_embed.reshape(1, H),
        half_move_embed,
        full_move_embed,
        repetition_embed,
        jnp.zeros((TROWS - ZERO_ROW, H), jnp.float32),
    ], axis=0)

    flat = _sc_gather(table.reshape(TROWS * H),
                      idxa.reshape(ROWS), idxb.reshape(ROWS))
    return flat.reshape(B, NT, H)


# speedup vs baseline: 5.8178x; 1.7639x over previous
import functools

import jax
import jax.numpy as jnp
from jax import lax
from jax.experimental import pallas as pl
from jax.experimental.pallas import tpu as pltpu
from jax.experimental.pallas import tpu_sc as plsc

B = 4096
H = 128
NT = 73
ROWS = B * NT
NW = 32
RPT = ROWS // NW
CHUNK = 128
NCH = RPT // CHUNK
NPAIR = (NCH - 1) // 2

OFF_PIECE = 0
OFF_SIDE = 13
OFF_CASTLE = 15
OFF_NOC = 19
OFF_POS = 20
OFF_NOEP = 84
OFF_HALF = 85
OFF_FULL = 185
OFF_REP = 441
ZERO_ROW = 444
TROWS = 448


def _prep_body(piece_ref, side_ref, castle_ref, ep_ref, hm_ref, fm_ref, rep_ref,
               idxp_ref):
    board_a = piece_ref[...] + OFF_PIECE
    side_a = side_ref[...] + OFF_SIDE
    cast_a = jnp.where(castle_ref[...] > 0,
                       lax.broadcasted_iota(jnp.int32, (B, 4), 1) + OFF_CASTLE,
                       OFF_NOC)
    ep = ep_ref[...]
    ep_a = jnp.where(ep < 64, ep + OFF_POS, OFF_NOEP)
    hm_a = jnp.clip(hm_ref[...], 0, 99) + OFF_HALF
    fm_a = jnp.clip(fm_ref[...], 1, 256) - 1 + OFF_FULL
    rep_a = jnp.clip(rep_ref[...] - 1, 0, 2) + OFF_REP
    idxa = jnp.concatenate(
        [board_a, side_a, cast_a, ep_a, hm_a, fm_a, rep_a], axis=1)
    board_b = lax.broadcasted_iota(jnp.int32, (B, 64), 1) + OFF_POS
    idxb = jnp.concatenate(
        [board_b, jnp.full((B, 9), ZERO_ROW, jnp.int32)], axis=1)
    idxp_ref[...] = idxa * 65536 + idxb


_prep = pl.pallas_call(
    _prep_body,
    out_shape=jax.ShapeDtypeStruct((B, NT), jnp.int32),
)


NB = CHUNK // 16


def _sc_body(table_hbm, idxp_hbm, out_hbm,
             table_v, idxp_v, rows0, rows1, ss0, ss1):
    wid = lax.axis_index("s") * 2 + lax.axis_index("c")
    base0 = wid * RPT

    pltpu.sync_copy(table_hbm, table_v)
    pltpu.sync_copy(idxp_hbm.at[pl.ds(base0, RPT)], idxp_v.at[pl.ds(0, RPT)])

    def compute_chunk(c, buf):
        @plsc.parallel_loop(0, CHUNK, unroll=4)
        def row(i):
            vp = idxp_v[pl.ds(c * CHUNK + i, 16)]
            p = vp[0]
            a = (p >> 16) * H
            b = (p & 0xFFFF) * H
            o = i * H
            sums = [table_v[pl.ds(a + j * 16, 16)]
                    + table_v[pl.ds(b + j * 16, 16)]
                    for j in range(H // 16)]
            for j in range(H // 16):
                buf[pl.ds(o + j * 16, 16)] = sums[j]

    def store(c, buf, sem):
        return pltpu.async_copy(
            buf, out_hbm.at[pl.ds((base0 + c * CHUNK) * H, CHUNK * H)], sem)

    def pair(g, carry):
        c0 = 2 * g
        compute_chunk(c0, rows0)
        s0 = store(c0, rows0, ss0)
        compute_chunk(c0 + 1, rows1)
        s0.wait()
        store(c0 + 1, rows1, ss1).wait()
        return carry

    lax.fori_loop(0, (NCH - 1) // 2, pair, 0)
    compute_chunk(NCH - 1, rows0)
    store(NCH - 1, rows0, ss0).wait()


_sc_gather = pl.kernel(
    _sc_body,
    out_type=jax.ShapeDtypeStruct((ROWS * H,), jnp.float32),
    mesh=plsc.VectorSubcoreMesh(core_axis_name="c", subcore_axis_name="s"),
    compiler_params=pltpu.CompilerParams(needs_layout_passes=False),
    scratch_types=(
        pltpu.VMEM((TROWS * H,), jnp.float32),
        pltpu.VMEM((RPT + 16,), jnp.int32),
        pltpu.VMEM((CHUNK * H,), jnp.float32),
        pltpu.VMEM((CHUNK * H,), jnp.float32),
        pltpu.SemaphoreType.DMA,
        pltpu.SemaphoreType.DMA,
    ),
)


def kernel(piece_indices, side_idx, castling_flags, en_passant_idx, halfmove,
           fullmove, repetitions, side_embed, castling_embed_K, castling_embed_Q,
           castling_embed_k, castling_embed_q, no_castling_embed, piece_embed,
           no_en_passant_embed, half_move_embed, full_move_embed,
           repetition_embed, pos_embed):
    i32 = jnp.int32
    idxp = _prep(
        piece_indices.astype(i32),
        side_idx.astype(i32).reshape(B, 1),
        castling_flags.astype(i32),
        en_passant_idx.astype(i32).reshape(B, 1),
        halfmove.astype(i32).reshape(B, 1),
        fullmove.astype(i32).reshape(B, 1),
        repetitions.astype(i32).reshape(B, 1))

    table = jnp.concatenate([
        piece_embed,
        side_embed,
        castling_embed_K.reshape(1, H),
        castling_embed_Q.reshape(1, H),
        castling_embed_k.reshape(1, H),
        castling_embed_q.reshape(1, H),
        no_castling_embed.reshape(1, H),
        pos_embed,
        no_en_passant_embed.reshape(1, H),
        half_move_embed,
        full_move_embed,
        repetition_embed,
        jnp.zeros((TROWS - ZERO_ROW, H), jnp.float32),
    ], axis=0)

    flat = _sc_gather(table.reshape(TROWS * H), idxp.reshape(ROWS))
    return flat.reshape(B, NT, H)


# speedup vs baseline: 8.7233x; 1.4994x over previous
import functools

import jax
import jax.numpy as jnp
from jax import lax
from jax.experimental import pallas as pl
from jax.experimental.pallas import tpu as pltpu
from jax.experimental.pallas import tpu_sc as plsc

B = 4096
H = 128
NT = 73
ROWS = B * NT
NW = 32
RPT = ROWS // NW
CHUNK = 128
NCH = RPT // CHUNK
NPAIR = (NCH - 1) // 2

OFF_PIECE = 0
OFF_SIDE = 13
OFF_CASTLE = 15
OFF_NOC = 19
OFF_POS = 20
OFF_NOEP = 84
OFF_HALF = 85
OFF_FULL = 185
OFF_REP = 441
ZERO_ROW = 444
TROWS = 448


def _prep_body(piece_ref, side_ref, castle_ref, ep_ref, hm_ref, fm_ref, rep_ref,
               idxp_ref):
    board_a = piece_ref[...] + OFF_PIECE
    side_a = side_ref[...] + OFF_SIDE
    cast_a = jnp.where(castle_ref[...] > 0,
                       lax.broadcasted_iota(jnp.int32, (B, 4), 1) + OFF_CASTLE,
                       OFF_NOC)
    ep = ep_ref[...]
    ep_a = jnp.where(ep < 64, ep + OFF_POS, OFF_NOEP)
    hm_a = jnp.clip(hm_ref[...], 0, 99) + OFF_HALF
    fm_a = jnp.clip(fm_ref[...], 1, 256) - 1 + OFF_FULL
    rep_a = jnp.clip(rep_ref[...] - 1, 0, 2) + OFF_REP
    idxa = jnp.concatenate(
        [board_a, side_a, cast_a, ep_a, hm_a, fm_a, rep_a], axis=1)
    board_b = lax.broadcasted_iota(jnp.int32, (B, 64), 1) + OFF_POS
    idxb = jnp.concatenate(
        [board_b, jnp.full((B, 9), ZERO_ROW, jnp.int32)], axis=1)
    idxp_ref[...] = idxa * 65536 + idxb


_prep = pl.pallas_call(
    _prep_body,
    out_shape=jax.ShapeDtypeStruct((B, NT), jnp.int32),
)


BPT = B // NW


def _sc_body(table_hbm, idxp_hbm, out_hbm,
             table_v, idxp_v, rows0, rows1, ss0, ss1):
    wid = lax.axis_index("s") * 2 + lax.axis_index("c")
    base0 = wid * RPT

    pltpu.sync_copy(table_hbm, table_v)
    pltpu.sync_copy(idxp_hbm.at[pl.ds(base0, RPT)], idxp_v.at[pl.ds(0, RPT)])

    def compute_board(bd, buf):
        @plsc.parallel_loop(0, NT, unroll=4)
        def row(i):
            vp = idxp_v[pl.ds(bd * NT + i, 16)]
            p = vp[0]
            a = (p >> 16) * H
            b = (p & 0xFFFF) * H
            sums = [table_v[pl.ds(a + j * 16, 16)]
                    + table_v[pl.ds(b + j * 16, 16)]
                    for j in range(H // 16)]
            for j in range(H // 16):
                buf[i, pl.ds(j * 16, 16)] = sums[j]

    def store(bd, buf, sem):
        return pltpu.async_copy(buf, out_hbm.at[wid * BPT + bd], sem)

    def pair(g, carry):
        b0 = 2 * g
        compute_board(b0, rows0)
        s0 = store(b0, rows0, ss0)
        compute_board(b0 + 1, rows1)
        s0.wait()
        store(b0 + 1, rows1, ss1).wait()
        return carry

    lax.fori_loop(0, BPT // 2, pair, 0)


_sc_gather = pl.kernel(
    _sc_body,
    out_type=jax.ShapeDtypeStruct((B, NT, H), jnp.float32),
    mesh=plsc.VectorSubcoreMesh(core_axis_name="c", subcore_axis_name="s"),
    compiler_params=pltpu.CompilerParams(needs_layout_passes=False),
    scratch_types=(
        pltpu.VMEM((TROWS * H,), jnp.float32),
        pltpu.VMEM((RPT + 16,), jnp.int32),
        pltpu.VMEM((NT, H), jnp.float32),
        pltpu.VMEM((NT, H), jnp.float32),
        pltpu.SemaphoreType.DMA,
        pltpu.SemaphoreType.DMA,
    ),
)


def kernel(piece_indices, side_idx, castling_flags, en_passant_idx, halfmove,
           fullmove, repetitions, side_embed, castling_embed_K, castling_embed_Q,
           castling_embed_k, castling_embed_q, no_castling_embed, piece_embed,
           no_en_passant_embed, half_move_embed, full_move_embed,
           repetition_embed, pos_embed):
    i32 = jnp.int32
    idxp = _prep(
        piece_indices.astype(i32),
        side_idx.astype(i32).reshape(B, 1),
        castling_flags.astype(i32),
        en_passant_idx.astype(i32).reshape(B, 1),
        halfmove.astype(i32).reshape(B, 1),
        fullmove.astype(i32).reshape(B, 1),
        repetitions.astype(i32).reshape(B, 1))

    table = jnp.concatenate([
        piece_embed,
        side_embed,
        castling_embed_K.reshape(1, H),
        castling_embed_Q.reshape(1, H),
        castling_embed_k.reshape(1, H),
        castling_embed_q.reshape(1, H),
        no_castling_embed.reshape(1, H),
        pos_embed,
        no_en_passant_embed.reshape(1, H),
        half_move_embed,
        full_move_embed,
        repetition_embed,
        jnp.zeros((TROWS - ZERO_ROW, H), jnp.float32),
    ], axis=0)

    return _sc_gather(table.reshape(TROWS * H), idxp.reshape(ROWS))


# speedup vs baseline: 9.3071x; 1.0669x over previous
import jax
import jax.numpy as jnp
from jax import lax
from jax.experimental import pallas as pl
from jax.experimental.pallas import tpu as pltpu
from jax.experimental.pallas import tpu_sc as plsc

B = 4096
H = 128
NT = 73
ROWS = B * NT
NW = 32
RPT = ROWS // NW
BPT = B // NW

OFF_PIECE = 0
OFF_SIDE = 13
OFF_CASTLE = 15
OFF_NOC = 19
OFF_POS = 20
OFF_NOEP = 84
OFF_HALF = 85
OFF_FULL = 185
OFF_REP = 441
ZERO_ROW = 444
TROWS = 448
PACK = 65536


def _sc_body(table_hbm, piece_hbm, side_hbm, castle_hbm, ep_hbm, hm_hbm,
             fm_hbm, rep_hbm, out_hbm,
             table_v, idxp_v, piece_v, side_v, castle_v, ep_v, hm_v, rep_v,
             fm_v, rows0, rows1, ss0, ss1):
    wid = lax.axis_index("s") * 2 + lax.axis_index("c")

    pltpu.sync_copy(table_hbm, table_v)
    pltpu.sync_copy(piece_hbm.at[pl.ds(wid * (BPT * 64), BPT * 64)], piece_v)
    pltpu.sync_copy(castle_hbm.at[pl.ds(wid * (BPT * 4), BPT * 4)], castle_v)
    pltpu.sync_copy(side_hbm.at[pl.ds(wid * BPT, BPT)], side_v)
    pltpu.sync_copy(ep_hbm.at[pl.ds(wid * BPT, BPT)], ep_v)
    pltpu.sync_copy(hm_hbm.at[pl.ds(wid * BPT, BPT)], hm_v)
    pltpu.sync_copy(fm_hbm.at[pl.ds(wid * BPT, BPT)], fm_v)
    pltpu.sync_copy(rep_hbm.at[pl.ds(wid * BPT, BPT)], rep_v)

    iota16 = jnp.arange(16, dtype=jnp.int32)

    @plsc.parallel_loop(0, BPT, unroll=4)
    def board_idx(bd):
        for v in range(4):
            p = piece_v[pl.ds(bd * 64 + v * 16, 16)]
            idxp_v[pl.ds(bd * NT + v * 16, 16)] = (
                p * PACK + (OFF_POS + v * 16) + iota16)

    @plsc.parallel_loop(0, BPT // 16, unroll=2)
    def extra_idx(g):
        bsel = pl.ds(g * 16, 16)
        pos0 = (g * 16 + iota16) * NT
        side_t = (side_v[bsel] + OFF_SIDE) * PACK + ZERO_ROW
        plsc.store_scatter(idxp_v, [pos0 + 64], side_t)
        for i in range(4):
            cf = plsc.load_gather(castle_v, [(g * 16 + iota16) * 4 + i])
            ct = jnp.where(cf > 0, OFF_CASTLE + i, OFF_NOC) * PACK + ZERO_ROW
            plsc.store_scatter(idxp_v, [pos0 + 65 + i], ct)
        ep = ep_v[bsel]
        ep_t = jnp.where(ep < 64, ep + OFF_POS, OFF_NOEP) * PACK + ZERO_ROW
        plsc.store_scatter(idxp_v, [pos0 + 69], ep_t)
        hm_t = (jnp.clip(hm_v[bsel], 0, 99) + OFF_HALF) * PACK + ZERO_ROW
        plsc.store_scatter(idxp_v, [pos0 + 70], hm_t)
        fm_t = (jnp.clip(fm_v[bsel], 1, 256) - 1 + OFF_FULL) * PACK + ZERO_ROW
        plsc.store_scatter(idxp_v, [pos0 + 71], fm_t)
        rep_t = (jnp.clip(rep_v[bsel] - 1, 0, 2) + OFF_REP) * PACK + ZERO_ROW
        plsc.store_scatter(idxp_v, [pos0 + 72], rep_t)

    def compute_board(bd, buf):
        @plsc.parallel_loop(0, NT, unroll=4)
        def row(i):
            vp = idxp_v[pl.ds(bd * NT + i, 16)]
            p = vp[0]
            a = (p >> 16) * H
            b = (p & 0xFFFF) * H
            sums = [table_v[pl.ds(a + j * 16, 16)]
                    + table_v[pl.ds(b + j * 16, 16)]
                    for j in range(H // 16)]
            for j in range(H // 16):
                buf[i, pl.ds(j * 16, 16)] = sums[j]

    def store(bd, buf, sem):
        return pltpu.async_copy(buf, out_hbm.at[wid * BPT + bd], sem)

    def pair(g, carry):
        b0 = 2 * g
        compute_board(b0, rows0)
        s0 = store(b0, rows0, ss0)
        compute_board(b0 + 1, rows1)
        s0.wait()
        store(b0 + 1, rows1, ss1).wait()
        return carry

    lax.fori_loop(0, BPT // 2, pair, 0)


_sc_gather = pl.kernel(
    _sc_body,
    out_type=jax.ShapeDtypeStruct((B, NT, H), jnp.float32),
    mesh=plsc.VectorSubcoreMesh(core_axis_name="c", subcore_axis_name="s"),
    compiler_params=pltpu.CompilerParams(needs_layout_passes=False),
    scratch_types=(
        pltpu.VMEM((TROWS * H,), jnp.float32),
        pltpu.VMEM((RPT + 16,), jnp.int32),
        pltpu.VMEM((BPT * 64,), jnp.int32),
        pltpu.VMEM((BPT,), jnp.int32),
        pltpu.VMEM((BPT * 4,), jnp.int32),
        pltpu.VMEM((BPT,), jnp.int32),
        pltpu.VMEM((BPT,), jnp.int32),
        pltpu.VMEM((BPT,), jnp.int32),
        pltpu.VMEM((BPT,), jnp.int32),
        pltpu.VMEM((NT, H), jnp.float32),
        pltpu.VMEM((NT, H), jnp.float32),
        pltpu.SemaphoreType.DMA,
        pltpu.SemaphoreType.DMA,
    ),
)


def kernel(piece_indices, side_idx, castling_flags, en_passant_idx, halfmove,
           fullmove, repetitions, side_embed, castling_embed_K, castling_embed_Q,
           castling_embed_k, castling_embed_q, no_castling_embed, piece_embed,
           no_en_passant_embed, half_move_embed, full_move_embed,
           repetition_embed, pos_embed):
    i32 = jnp.int32

    table = jnp.concatenate([
        piece_embed,
        side_embed,
        castling_embed_K.reshape(1, H),
        castling_embed_Q.reshape(1, H),
        castling_embed_k.reshape(1, H),
        castling_embed_q.reshape(1, H),
        no_castling_embed.reshape(1, H),
        pos_embed,
        no_en_passant_embed.reshape(1, H),
        half_move_embed,
        full_move_embed,
        repetition_embed,
        jnp.zeros((TROWS - ZERO_ROW, H), jnp.float32),
    ], axis=0)

    return _sc_gather(
        table.reshape(TROWS * H),
        piece_indices.astype(i32).reshape(B * 64),
        side_idx.astype(i32),
        castling_flags.astype(i32).reshape(B * 4),
        en_passant_idx.astype(i32),
        halfmove.astype(i32),
        fullmove.astype(i32),
        repetitions.astype(i32))


# speedup vs baseline: 9.8805x; 1.0616x over previous
import jax
import jax.numpy as jnp
from jax import lax
from jax.experimental import pallas as pl
from jax.experimental.pallas import tpu as pltpu
from jax.experimental.pallas import tpu_sc as plsc

B = 4096
H = 128
NT = 73
ROWS = B * NT
NW = 32
RPT = ROWS // NW
BPT = B // NW

OFF_PIECE = 0
OFF_SIDE = 13
OFF_CASTLE = 15
OFF_NOC = 19
OFF_POS = 20
OFF_NOEP = 84
OFF_HALF = 85
OFF_FULL = 185
OFF_REP = 441
ZERO_ROW = 444
TROWS = 448
PACK = 65536


def _sc_body(table_hbm, piece_hbm, side_hbm, castle_hbm, ep_hbm, hm_hbm,
             fm_hbm, rep_hbm, out_hbm,
             table_v, idxp_v, piece_v, side_v, castle_v, ep_v, hm_v, rep_v,
             fm_v, rows0, rows1, rows2, rows3, ss0, ss1, ss2, ss3):
    wid = lax.axis_index("s") * 2 + lax.axis_index("c")

    tcopy = pltpu.async_copy(table_hbm, table_v, ss0)
    pcopy = pltpu.async_copy(
        piece_hbm.at[pl.ds(wid * (BPT * 64), BPT * 64)], piece_v, ss1)
    pltpu.sync_copy(castle_hbm.at[pl.ds(wid * (BPT * 4), BPT * 4)], castle_v)
    pltpu.sync_copy(side_hbm.at[pl.ds(wid * BPT, BPT)], side_v)
    pltpu.sync_copy(ep_hbm.at[pl.ds(wid * BPT, BPT)], ep_v)
    pltpu.sync_copy(hm_hbm.at[pl.ds(wid * BPT, BPT)], hm_v)
    pltpu.sync_copy(fm_hbm.at[pl.ds(wid * BPT, BPT)], fm_v)
    pltpu.sync_copy(rep_hbm.at[pl.ds(wid * BPT, BPT)], rep_v)
    pcopy.wait()

    iota16 = jnp.arange(16, dtype=jnp.int32)

    @plsc.parallel_loop(0, BPT, unroll=4)
    def board_idx(bd):
        for v in range(4):
            p = piece_v[pl.ds(bd * 64 + v * 16, 16)]
            idxp_v[pl.ds(bd * NT + v * 16, 16)] = (
                p * PACK + (OFF_POS + v * 16) + iota16)

    @plsc.parallel_loop(0, BPT // 16, unroll=2)
    def extra_idx(g):
        bsel = pl.ds(g * 16, 16)
        pos0 = (g * 16 + iota16) * NT
        side_t = (side_v[bsel] + OFF_SIDE) * PACK + ZERO_ROW
        plsc.store_scatter(idxp_v, [pos0 + 64], side_t)
        for i in range(4):
            cf = plsc.load_gather(castle_v, [(g * 16 + iota16) * 4 + i])
            ct = jnp.where(cf > 0, OFF_CASTLE + i, OFF_NOC) * PACK + ZERO_ROW
            plsc.store_scatter(idxp_v, [pos0 + 65 + i], ct)
        ep = ep_v[bsel]
        ep_t = jnp.where(ep < 64, ep + OFF_POS, OFF_NOEP) * PACK + ZERO_ROW
        plsc.store_scatter(idxp_v, [pos0 + 69], ep_t)
        hm_t = (jnp.clip(hm_v[bsel], 0, 99) + OFF_HALF) * PACK + ZERO_ROW
        plsc.store_scatter(idxp_v, [pos0 + 70], hm_t)
        fm_t = (jnp.clip(fm_v[bsel], 1, 256) - 1 + OFF_FULL) * PACK + ZERO_ROW
        plsc.store_scatter(idxp_v, [pos0 + 71], fm_t)
        rep_t = (jnp.clip(rep_v[bsel] - 1, 0, 2) + OFF_REP) * PACK + ZERO_ROW
        plsc.store_scatter(idxp_v, [pos0 + 72], rep_t)

    tcopy.wait()

    def compute_board(bd, buf):
        @plsc.parallel_loop(0, NT, unroll=4)
        def row(i):
            vp = idxp_v[pl.ds(bd * NT + i, 16)]
            p = vp[0]
            a = (p >> 16) * H
            b = (p & 0xFFFF) * H
            sums = [table_v[pl.ds(a + j * 16, 16)]
                    + table_v[pl.ds(b + j * 16, 16)]
                    for j in range(H // 16)]
            for j in range(H // 16):
                buf[i, pl.ds(j * 16, 16)] = sums[j]

    def store(bd, buf, sem):
        return pltpu.async_copy(buf, out_hbm.at[wid * BPT + bd], sem)

    def quad(g, carry):
        b0 = 4 * g
        compute_board(b0, rows0)
        sa = store(b0, rows0, ss0)
        compute_board(b0 + 1, rows1)
        sb = store(b0 + 1, rows1, ss1)
        compute_board(b0 + 2, rows2)
        sa.wait()
        sc = store(b0 + 2, rows2, ss2)
        compute_board(b0 + 3, rows3)
        sb.wait()
        sd = store(b0 + 3, rows3, ss3)
        sc.wait()
        sd.wait()
        return carry

    lax.fori_loop(0, BPT // 4, quad, 0)


_sc_gather = pl.kernel(
    _sc_body,
    out_type=jax.ShapeDtypeStruct((B, NT, H), jnp.float32),
    mesh=plsc.VectorSubcoreMesh(core_axis_name="c", subcore_axis_name="s"),
    compiler_params=pltpu.CompilerParams(needs_layout_passes=False),
    scratch_types=(
        pltpu.VMEM((TROWS * H,), jnp.float32),
        pltpu.VMEM((RPT + 16,), jnp.int32),
        pltpu.VMEM((BPT * 64,), jnp.int32),
        pltpu.VMEM((BPT,), jnp.int32),
        pltpu.VMEM((BPT * 4,), jnp.int32),
        pltpu.VMEM((BPT,), jnp.int32),
        pltpu.VMEM((BPT,), jnp.int32),
        pltpu.VMEM((BPT,), jnp.int32),
        pltpu.VMEM((BPT,), jnp.int32),
        pltpu.VMEM((NT, H), jnp.float32),
        pltpu.VMEM((NT, H), jnp.float32),
        pltpu.VMEM((NT, H), jnp.float32),
        pltpu.VMEM((NT, H), jnp.float32),
        pltpu.SemaphoreType.DMA,
        pltpu.SemaphoreType.DMA,
        pltpu.SemaphoreType.DMA,
        pltpu.SemaphoreType.DMA,
    ),
)


def kernel(piece_indices, side_idx, castling_flags, en_passant_idx, halfmove,
           fullmove, repetitions, side_embed, castling_embed_K, castling_embed_Q,
           castling_embed_k, castling_embed_q, no_castling_embed, piece_embed,
           no_en_passant_embed, half_move_embed, full_move_embed,
           repetition_embed, pos_embed):
    i32 = jnp.int32

    table = jnp.concatenate([
        piece_embed,
        side_embed,
        castling_embed_K.reshape(1, H),
        castling_embed_Q.reshape(1, H),
        castling_embed_k.reshape(1, H),
        castling_embed_q.reshape(1, H),
        no_castling_embed.reshape(1, H),
        pos_embed,
        no_en_passant_embed.reshape(1, H),
        half_move_embed,
        full_move_embed,
        repetition_embed,
        jnp.zeros((TROWS - ZERO_ROW, H), jnp.float32),
    ], axis=0)

    return _sc_gather(
        table.reshape(TROWS * H),
        piece_indices.astype(i32).reshape(B * 64),
        side_idx.astype(i32),
        castling_flags.astype(i32).reshape(B * 4),
        en_passant_idx.astype(i32),
        halfmove.astype(i32),
        fullmove.astype(i32),
        repetitions.astype(i32))
